# Initial kernel scaffold; baseline (speedup 1.0000x reference)
import jax
import jax.numpy as jnp
from jax.experimental import pallas as pl


def kernel(x, G, W1, b1, W2, b2):
    raise NotImplementedError("write your pallas kernel here")


# speedup vs baseline: 1.0001x; 1.0001x over previous
import jax
import jax.numpy as jnp
from jax.experimental import pallas as pl

_BR = 400


def _lin_kernel(x_ref, w_ref, b_ref, o_ref):
    o_ref[:, :] = (
        jnp.dot(x_ref[:, :], w_ref[:, :], preferred_element_type=jnp.float32)
        + b_ref[:, :]
    )


def _pass1_kernel(g_ref, y1_ref, y1b_ref, w2_ref, b2_ref, z_ref):
    x1 = jnp.dot(g_ref[:, :], y1_ref[:, :], preferred_element_type=jnp.float32)
    z_ref[:, :] = (
        y1b_ref[:, :]
        + jnp.dot(x1, w2_ref[:, :], preferred_element_type=jnp.float32)
        + b2_ref[:, :]
    )


def _pass2_kernel(g_ref, z_ref, o_ref):
    o_ref[:, :] = 0.5 * jnp.dot(
        g_ref[:, :], z_ref[:, :], preferred_element_type=jnp.float32
    )


def kernel(x, G, W1, b1, W2, b2):
    N, D = x.shape
    b1r = b1.reshape(1, D)
    b2r = b2.reshape(1, D)

    y1 = pl.pallas_call(
        _lin_kernel,
        out_shape=jax.ShapeDtypeStruct((N, D), jnp.float32),
    )(x, W1, b1r)

    grid = (N // _BR,)
    z = pl.pallas_call(
        _pass1_kernel,
        grid=grid,
        in_specs=[
            pl.BlockSpec((_BR, N), lambda i: (i, 0)),
            pl.BlockSpec((N, D), lambda i: (0, 0)),
            pl.BlockSpec((_BR, D), lambda i: (i, 0)),
            pl.BlockSpec((D, D), lambda i: (0, 0)),
            pl.BlockSpec((1, D), lambda i: (0, 0)),
        ],
        out_specs=pl.BlockSpec((_BR, D), lambda i: (i, 0)),
        out_shape=jax.ShapeDtypeStruct((N, D), jnp.float32),
    )(G, y1, y1, W2, b2r)

    out = pl.pallas_call(
        _pass2_kernel,
        grid=grid,
        in_specs=[
            pl.BlockSpec((_BR, N), lambda i: (i, 0)),
            pl.BlockSpec((N, D), lambda i: (0, 0)),
        ],
        out_specs=pl.BlockSpec((_BR, D), lambda i: (i, 0)),
        out_shape=jax.ShapeDtypeStruct((N, D), jnp.float32),
    )(G, z)

    return out
